# baseline (speedup 1.0000x reference)
import jax
import jax.numpy as jnp
from jax.experimental import pallas as pl
from jax.experimental.pallas import tpu as pltpu
from jax.experimental.pallas import tpu_sc as plsc

_B = 8
_N = 4096
_D = 512
_C = 1024
_NU = 0.001
_K = 1
_TB = 4096
_NBLK = (_B * _N) // _TB
_LOSS_SCALE = 1.0 / (_NU * _B * _N * _K)

_NCORES = 2
_NSUB = 16
_NW = _NCORES * _NSUB
_ROWS_W = (_B * _N) // _NW


def _dist_kernel(e_ref, ct_ref, r_ref, score_ref, loss_ref, ctm2_ref, cen_ref):
    i = pl.program_id(0)

    @pl.when(i == 0)
    def _prep():
        ct = ct_ref[...]
        ctm2_ref[...] = (-2.0 * ct).astype(jnp.bfloat16)
        cen_ref[...] = jnp.sum(ct * ct, axis=0, keepdims=True)

    e = e_ref[...]
    p = jnp.dot(e.astype(jnp.bfloat16), ctm2_ref[...],
                preferred_element_type=jnp.float32)
    d2 = cen_ref[...] + p
    m = jnp.min(d2, axis=1, keepdims=True)
    feat = jnp.sum(e * e, axis=1, keepdims=True)
    dist = jnp.sqrt(feat + m)
    score_ref[...] = dist
    part = jnp.sum(jnp.maximum(dist - r_ref[0] * r_ref[0], 0.0))

    @pl.when(i == 0)
    def _init():
        loss_ref[0, 0] = 0.0

    loss_ref[0, 0] += part

    @pl.when(i == _NBLK - 1)
    def _finish():
        loss_ref[0, 0] = loss_ref[0, 0] * _LOSS_SCALE


def _copy_kernel(src_ref, dst_ref):
    c = jax.lax.axis_index("c")
    s = jax.lax.axis_index("s")
    wid = s * _NCORES + c
    pltpu.sync_copy(src_ref.at[pl.ds(wid, 1)], dst_ref.at[pl.ds(wid, 1)])


def kernel(embeds, centroids, r):
    e2d = embeds.reshape(_B * _N, _D)
    ct = centroids.T

    e_wide = embeds.reshape(_NW, (_B * _N * _D) // _NW)
    e_out = pl.kernel(
        _copy_kernel,
        mesh=plsc.VectorSubcoreMesh(
            core_axis_name="c", subcore_axis_name="s", num_cores=_NCORES
        ),
        out_type=jax.ShapeDtypeStruct((_NW, (_B * _N * _D) // _NW),
                                      jnp.float32),
    )(e_wide)

    score_flat, loss = pl.pallas_call(
        _dist_kernel,
        grid=(_NBLK,),
        in_specs=[
            pl.BlockSpec((_TB, _D), lambda i: (i, 0)),
            pl.BlockSpec((_D, _C), lambda i: (0, 0)),
            pl.BlockSpec(memory_space=pltpu.SMEM),
        ],
        out_specs=[
            pl.BlockSpec((_TB, 1), lambda i: (i, 0)),
            pl.BlockSpec(memory_space=pltpu.SMEM),
        ],
        out_shape=[
            jax.ShapeDtypeStruct((_B * _N, 1), jnp.float32),
            jax.ShapeDtypeStruct((1, 1), jnp.float32),
        ],
        scratch_shapes=[
            pltpu.VMEM((_D, _C), jnp.bfloat16),
            pltpu.VMEM((1, _C), jnp.float32),
        ],
        compiler_params=pltpu.CompilerParams(
            dimension_semantics=("arbitrary",),
        ),
    )(e2d, ct, r)

    h = 64
    score = score_flat.reshape(_B, 1, h, h)
    return (loss[0, 0], score, e_out.reshape(_B, _N, _D))


# speedup vs baseline: 29.7931x; 29.7931x over previous
import jax
import jax.numpy as jnp
from jax.experimental import pallas as pl
from jax.experimental.pallas import tpu as pltpu

_B = 8
_N = 4096
_D = 512
_C = 1024
_NU = 0.001
_K = 1
_TB = 4096
_NBLK = (_B * _N) // _TB
_LOSS_SCALE = 1.0 / (_NU * _B * _N * _K)


def _dist_kernel(e_ref, ct_ref, r_ref, score_ref, part_ref, eout_ref):
    ct = ct_ref[...]
    ctm2 = (-2.0 * ct).astype(jnp.bfloat16)
    cen = jnp.sum(ct * ct, axis=0, keepdims=True)

    e = e_ref[...]
    p = jnp.dot(e.astype(jnp.bfloat16), ctm2,
                preferred_element_type=jnp.float32)
    d2 = cen + p
    m = jnp.min(d2, axis=1, keepdims=True)
    feat = jnp.sum(e * e, axis=1, keepdims=True)
    dist = jnp.sqrt(feat + m)
    score_ref[...] = dist
    eout_ref[...] = e
    part = jnp.sum(jnp.maximum(dist - r_ref[0] * r_ref[0], 0.0))
    part_ref[...] = jnp.full((1, 1, 128), part, dtype=jnp.float32)


def kernel(embeds, centroids, r):
    e2d = embeds.reshape(_B * _N, _D)
    ct = centroids.T
    score_flat, parts, e_out = pl.pallas_call(
        _dist_kernel,
        grid=(_NBLK,),
        in_specs=[
            pl.BlockSpec((_TB, _D), lambda i: (i, 0)),
            pl.BlockSpec((_D, _C), lambda i: (0, 0)),
            pl.BlockSpec(memory_space=pltpu.SMEM),
        ],
        out_specs=[
            pl.BlockSpec((_TB, 1), lambda i: (i, 0)),
            pl.BlockSpec((1, 1, 128), lambda i: (i, 0, 0)),
            pl.BlockSpec((_TB, _D), lambda i: (i, 0)),
        ],
        out_shape=[
            jax.ShapeDtypeStruct((_B * _N, 1), jnp.float32),
            jax.ShapeDtypeStruct((_NBLK, 1, 128), jnp.float32),
            jax.ShapeDtypeStruct((_B * _N, _D), jnp.float32),
        ],
        compiler_params=pltpu.CompilerParams(
            dimension_semantics=("parallel",),
        ),
    )(e2d, ct, r)
    h = 64
    score = score_flat.reshape(_B, 1, h, h)
    loss = jnp.sum(parts[:, 0, 0]) * _LOSS_SCALE
    return (loss, score, e_out.reshape(_B, _N, _D))


# speedup vs baseline: 34.6750x; 1.1639x over previous
import jax
import jax.numpy as jnp
from jax.experimental import pallas as pl
from jax.experimental.pallas import tpu as pltpu

_B = 8
_N = 4096
_D = 512
_C = 1024
_NU = 0.001
_K = 1
_TB = 4096
_NBLK = (_B * _N) // _TB
_LOSS_SCALE = 1.0 / (_NU * _B * _N * _K)


def _dist_kernel(e_ref, ct_ref, r_ref, score_ref, loss_ref, eout_ref,
                 ctm2_ref, cen_ref):
    i = pl.program_id(0)

    @pl.when(i == 0)
    def _prep():
        c = ct_ref[...]
        ct = jnp.transpose(c)
        ctm2_ref[...] = (-2.0 * ct).astype(jnp.bfloat16)
        cen_ref[...] = jnp.sum(ct * ct, axis=0, keepdims=True)

    e = e_ref[...]
    p = jnp.dot(e.astype(jnp.bfloat16), ctm2_ref[...],
                preferred_element_type=jnp.float32)
    d2 = cen_ref[...] + p
    m = jnp.min(d2, axis=1, keepdims=True)
    feat = jnp.sum(e * e, axis=1, keepdims=True)
    dist = jnp.sqrt(feat + m)
    score_ref[...] = dist
    eout_ref[...] = e
    part = jnp.sum(jnp.maximum(dist - r_ref[0] * r_ref[0], 0.0))

    @pl.when(i == 0)
    def _init():
        loss_ref[0, 0] = 0.0

    loss_ref[0, 0] += part

    @pl.when(i == _NBLK - 1)
    def _finish():
        loss_ref[0, 0] = loss_ref[0, 0] * _LOSS_SCALE


def kernel(embeds, centroids, r):
    e2d = embeds.reshape(_B * _N, _D)
    score_flat, loss, e_out = pl.pallas_call(
        _dist_kernel,
        grid=(_NBLK,),
        in_specs=[
            pl.BlockSpec((_TB, _D), lambda i: (i, 0)),
            pl.BlockSpec((_C, _D), lambda i: (0, 0)),
            pl.BlockSpec(memory_space=pltpu.SMEM),
        ],
        out_specs=[
            pl.BlockSpec((_TB, 1), lambda i: (i, 0)),
            pl.BlockSpec(memory_space=pltpu.SMEM),
            pl.BlockSpec((_TB, _D), lambda i: (i, 0)),
        ],
        out_shape=[
            jax.ShapeDtypeStruct((_B * _N, 1), jnp.float32),
            jax.ShapeDtypeStruct((1, 1), jnp.float32),
            jax.ShapeDtypeStruct((_B * _N, _D), jnp.float32),
        ],
        scratch_shapes=[
            pltpu.VMEM((_D, _C), jnp.bfloat16),
            pltpu.VMEM((1, _C), jnp.float32),
        ],
        compiler_params=pltpu.CompilerParams(
            dimension_semantics=("arbitrary",),
        ),
    )(e2d, centroids, r)
    h = 64
    score = score_flat.reshape(_B, 1, h, h)
    return (loss[0, 0], score, e_out.reshape(_B, _N, _D))


# speedup vs baseline: 41.1888x; 1.1879x over previous
import jax
import jax.numpy as jnp
from jax.experimental import pallas as pl
from jax.experimental.pallas import tpu as pltpu

_B = 8
_N = 4096
_D = 512
_C = 1024
_NU = 0.001
_K = 1
_TB = 4096
_NBLK = (_B * _N) // _TB
_LOSS_SCALE = 1.0 / (_NU * _B * _N * _K)


def _dist_kernel(e_ref, ct_ref, r_ref, score_ref, loss_ref, eout_ref,
                 ctm2_ref, cen_ref):
    i = pl.program_id(0)

    @pl.when(i == 0)
    def _prep():
        c = ct_ref[...]
        ct = jnp.transpose(c)
        ctm2_ref[...] = (-2.0 * ct).astype(jnp.bfloat16)
        cen_ref[...] = jnp.sum(ct * ct, axis=0, keepdims=True)

    e = e_ref[...]
    p = jnp.dot(e.astype(jnp.bfloat16), ctm2_ref[...],
                preferred_element_type=jnp.float32)
    d2 = cen_ref[...] + p
    m = jnp.min(d2, axis=1, keepdims=True)
    feat = jnp.sum(e * e, axis=1, keepdims=True)
    dist = jnp.sqrt(feat + m)
    score_ref[...] = dist.reshape(1, 1, 64, 64)
    eout_ref[...] = e
    part = jnp.sum(jnp.maximum(dist - r_ref[0] * r_ref[0], 0.0))

    @pl.when(i == 0)
    def _init():
        loss_ref[0, 0] = 0.0

    loss_ref[0, 0] += part

    @pl.when(i == _NBLK - 1)
    def _finish():
        loss_ref[0, 0] = loss_ref[0, 0] * _LOSS_SCALE


def kernel(embeds, centroids, r):
    e2d = embeds.reshape(_B * _N, _D)
    score_flat, loss, e_out = pl.pallas_call(
        _dist_kernel,
        grid=(_NBLK,),
        in_specs=[
            pl.BlockSpec((_TB, _D), lambda i: (i, 0)),
            pl.BlockSpec((_C, _D), lambda i: (0, 0)),
            pl.BlockSpec(memory_space=pltpu.SMEM),
        ],
        out_specs=[
            pl.BlockSpec((1, 1, 64, 64), lambda i: (i, 0, 0, 0)),
            pl.BlockSpec(memory_space=pltpu.SMEM),
            pl.BlockSpec((_TB, _D), lambda i: (i, 0)),
        ],
        out_shape=[
            jax.ShapeDtypeStruct((_B, 1, 64, 64), jnp.float32),
            jax.ShapeDtypeStruct((1, 1), jnp.float32),
            jax.ShapeDtypeStruct((_B * _N, _D), jnp.float32),
        ],
        scratch_shapes=[
            pltpu.VMEM((_D, _C), jnp.bfloat16),
            pltpu.VMEM((1, _C), jnp.float32),
        ],
        compiler_params=pltpu.CompilerParams(
            dimension_semantics=("arbitrary",),
        ),
    )(e2d, centroids, r)
    return (loss[0, 0], score_flat, e_out.reshape(_B, _N, _D))
